# baseline (speedup 1.0000x reference)
import functools

import jax
import jax.numpy as jnp
from jax import lax
from jax.experimental import pallas as pl
from jax.experimental.pallas import tpu as pltpu
from jax.experimental.pallas import tpu_sc as plsc

N = 10000
D = 128
E = 320000
NC = 2
NS = 16
NW = NC * NS
BS = 128
EPW = 10240
STEPS = EPW // BS
EPAD = EPW * NW
NPAD = 10240
RPT = NPAD // NS
WINDOWS = 4
WSTEPS = STEPS // WINDOWS
WPAIRS = WSTEPS // 2

_mesh = plsc.VectorSubcoreMesh(core_axis_name="c", subcore_axis_name="s")


@functools.partial(
    pl.kernel,
    out_type=(jax.ShapeDtypeStruct((NPAD,), jnp.float32),
              jax.ShapeDtypeStruct((NPAD,), jnp.float32)),
    mesh=_mesh,
    scratch_types=[
        pltpu.VMEM((WSTEPS, BS), jnp.int32),
        pltpu.VMEM((BS,), jnp.float32),
        pltpu.VMEM_SHARED((NPAD,), jnp.float32),
    ],
)
def _sc_counts(dst3, zpad, out0, out1, idx_v, ones_v, cnt_sh):
    cid = lax.axis_index("c")
    sid = lax.axis_index("s")
    wid = cid * NS + sid
    pltpu.sync_copy(zpad.at[pl.ds(sid * RPT, RPT)], cnt_sh.at[pl.ds(sid * RPT, RPT)])
    for j in range(BS // 16):
        ones_v[pl.ds(j * 16, 16)] = jnp.ones((16,), jnp.float32)
    plsc.subcore_barrier()

    def body(i, carry):
        pltpu.sync_copy(ones_v, cnt_sh.at[idx_v.at[i]], add=True)
        return carry

    for h in range(WINDOWS):
        pltpu.sync_copy(dst3.at[wid, h], idx_v)
        lax.fori_loop(0, WSTEPS, body, 0)
    plsc.subcore_barrier()

    @pl.when(cid == 0)
    def _():
        pltpu.sync_copy(cnt_sh.at[pl.ds(sid * RPT, RPT)], out0.at[pl.ds(sid * RPT, RPT)])

    @pl.when(cid == 1)
    def _():
        pltpu.sync_copy(cnt_sh.at[pl.ds(sid * RPT, RPT)], out1.at[pl.ds(sid * RPT, RPT)])


@functools.partial(
    pl.kernel,
    out_type=(jax.ShapeDtypeStruct((NPAD, D), jnp.float32),
              jax.ShapeDtypeStruct((NPAD, D), jnp.float32)),
    mesh=_mesh,
    scratch_types=[
        pltpu.VMEM((WSTEPS, BS), jnp.int32),
        pltpu.VMEM((WSTEPS, BS), jnp.int32),
        pltpu.VMEM((BS, D), jnp.float32),
        pltpu.VMEM((BS, D), jnp.float32),
        pltpu.VMEM_SHARED((NPAD, D), jnp.float32),
        pltpu.SemaphoreType.DMA,
        pltpu.SemaphoreType.DMA,
        pltpu.SemaphoreType.DMA,
        pltpu.SemaphoreType.DMA,
    ],
)
def _sc_agg(y, src3, dst3, zrows, out0, out1, src_v, dst_v,
            b0, b1, acc_sh,
            g0, g1, s0, s1):
    cid = lax.axis_index("c")
    sid = lax.axis_index("s")
    wid = cid * NS + sid
    bufs = (b0, b1)
    gsems = (g0, g1)
    ssems = (s0, s1)
    pltpu.sync_copy(zrows.at[pl.ds(sid * RPT, RPT)], acc_sh.at[pl.ds(sid * RPT, RPT)])
    plsc.subcore_barrier()

    def gstart(step, k):
        pltpu.async_copy(y.at[src_v.at[step]], bufs[k], gsems[k])

    def gwait(step, k):
        pltpu.make_async_copy(y.at[src_v.at[step]], bufs[k], gsems[k]).wait()

    def sstart(step, k):
        pltpu.async_copy(bufs[k], acc_sh.at[dst_v.at[step]], ssems[k], add=True)

    def swait(step, k):
        pltpu.make_async_copy(bufs[k], acc_sh.at[dst_v.at[step]], ssems[k]).wait()

    def body(g, carry):
        sa = 2 * g
        sb = sa + 1
        gwait(sa, 0)
        sstart(sa, 0)
        gwait(sb, 1)
        sstart(sb, 1)

        @pl.when(g < WPAIRS - 1)
        def _():
            swait(sa, 0)
            gstart(sa + 2, 0)
            swait(sb, 1)
            gstart(sb + 2, 1)

        return carry

    for h in range(WINDOWS):
        pltpu.sync_copy(src3.at[wid, h], src_v)
        pltpu.sync_copy(dst3.at[wid, h], dst_v)
        gstart(0, 0)
        gstart(1, 1)
        lax.fori_loop(0, WPAIRS, body, 0)
        swait(WSTEPS - 2, 0)
        swait(WSTEPS - 1, 1)

    plsc.subcore_barrier()

    @pl.when(cid == 0)
    def _():
        pltpu.sync_copy(acc_sh.at[pl.ds(sid * RPT, RPT)], out0.at[pl.ds(sid * RPT, RPT)])

    @pl.when(cid == 1)
    def _():
        pltpu.sync_copy(acc_sh.at[pl.ds(sid * RPT, RPT)], out1.at[pl.ds(sid * RPT, RPT)])


RB = 1000


def _dense1_body(c0, c1, s0, s1, x, wl, bl, wr, h_out, invc_out):
    invc = 1.0 / jnp.maximum(c0[...] + c1[...], 1.0)
    mean = (s0[...] + s1[...]) * invc
    h = (jnp.dot(mean, wl[...], preferred_element_type=jnp.float32) + bl[...]
         + jnp.dot(x[...], wr[...], preferred_element_type=jnp.float32))
    h_out[...] = jnp.maximum(h, 0.0)
    invc_out[...] = invc


def _dense2_body(invc, s0, s1, h1, wl, bl, wr, w3l, h_out, y3_out):
    mean = (s0[...] + s1[...]) * invc[...]
    h = (jnp.dot(mean, wl[...], preferred_element_type=jnp.float32) + bl[...]
         + jnp.dot(h1[...], wr[...], preferred_element_type=jnp.float32))
    h = jnp.maximum(h, 0.0)
    h_out[...] = h
    y3_out[...] = jnp.dot(h, w3l[...], preferred_element_type=jnp.float32)


def _dense3_body(invc, s0, s1, h2, bl, wr, h_out):
    mean = (s0[...] + s1[...]) * invc[...]
    h = mean + bl[...] + jnp.dot(h2[...], wr[...], preferred_element_type=jnp.float32)
    h_out[...] = jnp.maximum(h, 0.0)


def _row_spec(w):
    return pl.BlockSpec((RB, w), lambda i: (i, 0))


def _full_spec(shape):
    return pl.BlockSpec(shape, lambda i: tuple(0 for _ in shape))


def kernel(x, W1l, b1, W1r, W2l, b2, W2r, W3l, b3, W3r, edge_index):
    npad_e = EPAD - E
    pad_iota = jnp.arange(npad_e, dtype=jnp.int32)
    src3 = jnp.concatenate(
        [edge_index[0], pad_iota % N]).reshape(NW, WINDOWS, WSTEPS, BS)
    dst3 = jnp.concatenate(
        [edge_index[1], N + pad_iota % (NPAD - N)]).reshape(NW, WINDOWS, WSTEPS, BS)
    zpad = jnp.zeros((NPAD,), jnp.float32)
    zrows = jnp.zeros((NPAD, D), jnp.float32)

    cnt0, cnt1 = _sc_counts(dst3, zpad)
    c0 = cnt0[:N, None]
    c1 = cnt1[:N, None]

    grid = (N // RB,)

    sa0, sa1 = _sc_agg(x, src3, dst3, zrows)
    h1, invc = pl.pallas_call(
        _dense1_body,
        grid=grid,
        in_specs=[_row_spec(1), _row_spec(1), _row_spec(D), _row_spec(D),
                  _row_spec(D), _full_spec((D, D)), _full_spec((1, D)),
                  _full_spec((D, D))],
        out_specs=[_row_spec(D), _row_spec(1)],
        out_shape=[jax.ShapeDtypeStruct((N, D), jnp.float32),
                   jax.ShapeDtypeStruct((N, 1), jnp.float32)],
    )(c0, c1, sa0, sa1, x, W1l, b1.reshape(1, D), W1r)

    sa0, sa1 = _sc_agg(h1, src3, dst3, zrows)
    h2, y3 = pl.pallas_call(
        _dense2_body,
        grid=grid,
        in_specs=[_row_spec(1), _row_spec(D), _row_spec(D), _row_spec(D),
                  _full_spec((D, 2 * D)), _full_spec((1, 2 * D)),
                  _full_spec((D, 2 * D)), _full_spec((2 * D, D))],
        out_specs=[_row_spec(2 * D), _row_spec(D)],
        out_shape=[jax.ShapeDtypeStruct((N, 2 * D), jnp.float32),
                   jax.ShapeDtypeStruct((N, D), jnp.float32)],
    )(invc, sa0, sa1, h1, W2l, b2.reshape(1, 2 * D), W2r, W3l)

    sa0, sa1 = _sc_agg(y3, src3, dst3, zrows)
    h3 = pl.pallas_call(
        _dense3_body,
        grid=grid,
        in_specs=[_row_spec(1), _row_spec(D), _row_spec(D), _row_spec(2 * D),
                  _full_spec((1, D)), _full_spec((2 * D, D))],
        out_specs=_row_spec(D),
        out_shape=jax.ShapeDtypeStruct((N, D), jnp.float32),
    )(invc, sa0, sa1, h2, b3.reshape(1, D), W3r)
    return h3


# speedup vs baseline: 1.2266x; 1.2266x over previous
import functools

import jax
import jax.numpy as jnp
from jax import lax
from jax.experimental import pallas as pl
from jax.experimental.pallas import tpu as pltpu
from jax.experimental.pallas import tpu_sc as plsc

N = 10000
D = 128
E = 320000
NC = 2
NS = 16
NW = NC * NS
BS = 128
EPW = 10240
STEPS = EPW // BS
EPAD = EPW * NW
NPAD = 10240
RPT = NPAD // NS
WINDOWS = 4
WSTEPS = STEPS // WINDOWS
WPAIRS = WSTEPS // 2

_mesh = plsc.VectorSubcoreMesh(core_axis_name="c", subcore_axis_name="s")


@functools.partial(
    pl.kernel,
    out_type=(jax.ShapeDtypeStruct((NPAD,), jnp.float32),
              jax.ShapeDtypeStruct((NPAD,), jnp.float32)),
    mesh=_mesh,
    scratch_types=[
        pltpu.VMEM((WSTEPS, BS), jnp.int32),
        pltpu.VMEM((BS,), jnp.float32),
        pltpu.VMEM_SHARED((NPAD,), jnp.float32),
    ],
)
def _sc_counts(dst3, zpad, out0, out1, idx_v, ones_v, cnt_sh):
    cid = lax.axis_index("c")
    sid = lax.axis_index("s")
    wid = cid * NS + sid
    pltpu.sync_copy(zpad.at[pl.ds(sid * RPT, RPT)], cnt_sh.at[pl.ds(sid * RPT, RPT)])
    for j in range(BS // 16):
        ones_v[pl.ds(j * 16, 16)] = jnp.ones((16,), jnp.float32)
    plsc.subcore_barrier()

    def body(i, carry):
        pltpu.sync_copy(ones_v, cnt_sh.at[idx_v.at[i]], add=True)
        return carry

    for h in range(WINDOWS):
        pltpu.sync_copy(dst3.at[wid, h], idx_v)
        lax.fori_loop(0, WSTEPS, body, 0)
    plsc.subcore_barrier()

    @pl.when(cid == 0)
    def _():
        pltpu.sync_copy(cnt_sh.at[pl.ds(sid * RPT, RPT)], out0.at[pl.ds(sid * RPT, RPT)])

    @pl.when(cid == 1)
    def _():
        pltpu.sync_copy(cnt_sh.at[pl.ds(sid * RPT, RPT)], out1.at[pl.ds(sid * RPT, RPT)])


@functools.partial(
    pl.kernel,
    out_type=(jax.ShapeDtypeStruct((NPAD, D), jnp.float32),
              jax.ShapeDtypeStruct((NPAD, D), jnp.float32)),
    mesh=_mesh,
    scratch_types=[
        pltpu.VMEM((WSTEPS, BS), jnp.int32),
        pltpu.VMEM((WSTEPS, BS), jnp.int32),
        pltpu.VMEM((BS, D), jnp.float32),
        pltpu.VMEM((BS, D), jnp.float32),
        pltpu.VMEM_SHARED((NPAD, D), jnp.float32),
        pltpu.SemaphoreType.DMA,
        pltpu.SemaphoreType.DMA,
        pltpu.SemaphoreType.DMA,
        pltpu.SemaphoreType.DMA,
    ],
)
def _sc_agg(y, src3, dst3, zrows, out0, out1, src_v, dst_v,
            b0, b1, acc_sh,
            g0, g1, s0, s1):
    cid = lax.axis_index("c")
    sid = lax.axis_index("s")
    wid = cid * NS + sid
    bufs = (b0, b1)
    gsems = (g0, g1)
    ssems = (s0, s1)
    pltpu.sync_copy(zrows.at[pl.ds(sid * RPT, RPT)], acc_sh.at[pl.ds(sid * RPT, RPT)])
    plsc.subcore_barrier()

    def gstart(step, k):
        pltpu.async_copy(y.at[src_v.at[step]], bufs[k], gsems[k])

    def gwait(step, k):
        pltpu.make_async_copy(y.at[src_v.at[step]], bufs[k], gsems[k]).wait()

    def sstart(step, k):
        pltpu.async_copy(bufs[k], acc_sh.at[dst_v.at[step]], ssems[k], add=True)

    def swait(step, k):
        pltpu.make_async_copy(bufs[k], acc_sh.at[dst_v.at[step]], ssems[k]).wait()

    def scat(step, k):
        pltpu.sync_copy(bufs[k], acc_sh.at[dst_v.at[step]], add=True)

    def body(g, carry):
        sa = 2 * g
        sb = sa + 1
        gstart(sb, 1)
        gwait(sa, 0)
        scat(sa, 0)

        @pl.when(g < WPAIRS - 1)
        def _():
            gstart(sa + 2, 0)

        gwait(sb, 1)
        scat(sb, 1)
        return carry

    for h in range(WINDOWS):
        pltpu.sync_copy(src3.at[wid, h], src_v)
        pltpu.sync_copy(dst3.at[wid, h], dst_v)
        gstart(0, 0)
        lax.fori_loop(0, WPAIRS, body, 0)

    plsc.subcore_barrier()

    @pl.when(cid == 0)
    def _():
        pltpu.sync_copy(acc_sh.at[pl.ds(sid * RPT, RPT)], out0.at[pl.ds(sid * RPT, RPT)])

    @pl.when(cid == 1)
    def _():
        pltpu.sync_copy(acc_sh.at[pl.ds(sid * RPT, RPT)], out1.at[pl.ds(sid * RPT, RPT)])


RB = 1000


def _dense1_body(c0, c1, s0, s1, x, wl, bl, wr, h_out, invc_out):
    invc = 1.0 / jnp.maximum(c0[...] + c1[...], 1.0)
    mean = (s0[...] + s1[...]) * invc
    h = (jnp.dot(mean, wl[...], preferred_element_type=jnp.float32) + bl[...]
         + jnp.dot(x[...], wr[...], preferred_element_type=jnp.float32))
    h_out[...] = jnp.maximum(h, 0.0)
    invc_out[...] = invc


def _dense2_body(invc, s0, s1, h1, wl, bl, wr, w3l, h_out, y3_out):
    mean = (s0[...] + s1[...]) * invc[...]
    h = (jnp.dot(mean, wl[...], preferred_element_type=jnp.float32) + bl[...]
         + jnp.dot(h1[...], wr[...], preferred_element_type=jnp.float32))
    h = jnp.maximum(h, 0.0)
    h_out[...] = h
    y3_out[...] = jnp.dot(h, w3l[...], preferred_element_type=jnp.float32)


def _dense3_body(invc, s0, s1, h2, bl, wr, h_out):
    mean = (s0[...] + s1[...]) * invc[...]
    h = mean + bl[...] + jnp.dot(h2[...], wr[...], preferred_element_type=jnp.float32)
    h_out[...] = jnp.maximum(h, 0.0)


def _row_spec(w):
    return pl.BlockSpec((RB, w), lambda i: (i, 0))


def _full_spec(shape):
    return pl.BlockSpec(shape, lambda i: tuple(0 for _ in shape))


def kernel(x, W1l, b1, W1r, W2l, b2, W2r, W3l, b3, W3r, edge_index):
    npad_e = EPAD - E
    pad_iota = jnp.arange(npad_e, dtype=jnp.int32)
    src3 = jnp.concatenate(
        [edge_index[0], pad_iota % N]).reshape(NW, WINDOWS, WSTEPS, BS)
    dst3 = jnp.concatenate(
        [edge_index[1], N + pad_iota % (NPAD - N)]).reshape(NW, WINDOWS, WSTEPS, BS)
    zpad = jnp.zeros((NPAD,), jnp.float32)
    zrows = jnp.zeros((NPAD, D), jnp.float32)

    cnt0, cnt1 = _sc_counts(dst3, zpad)
    c0 = cnt0[:N, None]
    c1 = cnt1[:N, None]

    grid = (N // RB,)

    sa0, sa1 = _sc_agg(x, src3, dst3, zrows)
    h1, invc = pl.pallas_call(
        _dense1_body,
        grid=grid,
        in_specs=[_row_spec(1), _row_spec(1), _row_spec(D), _row_spec(D),
                  _row_spec(D), _full_spec((D, D)), _full_spec((1, D)),
                  _full_spec((D, D))],
        out_specs=[_row_spec(D), _row_spec(1)],
        out_shape=[jax.ShapeDtypeStruct((N, D), jnp.float32),
                   jax.ShapeDtypeStruct((N, 1), jnp.float32)],
    )(c0, c1, sa0, sa1, x, W1l, b1.reshape(1, D), W1r)

    sa0, sa1 = _sc_agg(h1, src3, dst3, zrows)
    h2, y3 = pl.pallas_call(
        _dense2_body,
        grid=grid,
        in_specs=[_row_spec(1), _row_spec(D), _row_spec(D), _row_spec(D),
                  _full_spec((D, 2 * D)), _full_spec((1, 2 * D)),
                  _full_spec((D, 2 * D)), _full_spec((2 * D, D))],
        out_specs=[_row_spec(2 * D), _row_spec(D)],
        out_shape=[jax.ShapeDtypeStruct((N, 2 * D), jnp.float32),
                   jax.ShapeDtypeStruct((N, D), jnp.float32)],
    )(invc, sa0, sa1, h1, W2l, b2.reshape(1, 2 * D), W2r, W3l)

    sa0, sa1 = _sc_agg(y3, src3, dst3, zrows)
    h3 = pl.pallas_call(
        _dense3_body,
        grid=grid,
        in_specs=[_row_spec(1), _row_spec(D), _row_spec(D), _row_spec(2 * D),
                  _full_spec((1, D)), _full_spec((2 * D, D))],
        out_specs=_row_spec(D),
        out_shape=jax.ShapeDtypeStruct((N, D), jnp.float32),
    )(invc, sa0, sa1, h2, b3.reshape(1, D), W3r)
    return h3
